# baseline (speedup 1.0000x reference)
import functools

import jax
import jax.numpy as jnp
from jax import lax
from jax.experimental import pallas as pl
from jax.experimental.pallas import tpu as pltpu
from jax.experimental.pallas import tpu_sc as plsc

N = 10000
E = 320000
D = 128
DE = 16
DU = 32

NC = 2
NS = 16
NW = NC * NS
EPW = E // NW
CH = 80
NIT = EPW // CH
CH4 = 40
NIT4 = EPW // CH4
CPR = 624
ZB = 52


def _proj_body(x_ref, wsd_ref, wm_ref, xsd_ref, xm_ref):
    xb = x_ref[...]
    xsd_ref[...] = jnp.dot(xb, wsd_ref[...], preferred_element_type=jnp.float32)
    xm_ref[...] = jnp.dot(xb, wm_ref[...], preferred_element_type=jnp.float32)


def _edgeconst_body(ea_ref, wee_ref, u_ref, weu_ref, be_ref, c_ref):
    cu = jnp.dot(u_ref[...], weu_ref[...], preferred_element_type=jnp.float32)
    c_ref[...] = (jnp.dot(ea_ref[...], wee_ref[...],
                          preferred_element_type=jnp.float32) + cu + be_ref[...])


def _tmat_body(e_ref, w_ref, b_ref, t_ref):
    t_ref[...] = jnp.dot(e_ref[...], w_ref[...],
                         preferred_element_type=jnp.float32) + b_ref[...]


def _node_body(x_ref, p0m_ref, p1m_ref, pc_ref, u_ref,
               w2x_ref, w2a_ref, w2u_ref, b2_ref, nx_ref, s_ref):
    i = pl.program_id(0)
    cnt = jnp.maximum(jnp.sum(pc_ref[...], axis=1, keepdims=True), 1.0)
    agg = (p0m_ref[...] + p1m_ref[...]) / cnt
    nx = jnp.dot(x_ref[...], w2x_ref[...], preferred_element_type=jnp.float32)
    nx = nx + jnp.dot(agg, w2a_ref[...], preferred_element_type=jnp.float32)
    nx = nx + jnp.dot(u_ref[...], w2u_ref[...], preferred_element_type=jnp.float32)
    nx = jnp.maximum(nx + b2_ref[...], 0.0)
    nx_ref[...] = nx

    @pl.when(i == 0)
    def _():
        s_ref[...] = jnp.zeros_like(s_ref)

    s_ref[...] += jnp.sum(nx, axis=0, keepdims=True)


def _glob_body(u_ref, s_ref, wgu_ref, wgx_ref, bg_ref, out_ref):
    m = s_ref[...] * (1.0 / N)
    g = (jnp.dot(u_ref[...], wgu_ref[...], preferred_element_type=jnp.float32)
         + jnp.dot(m, wgx_ref[...], preferred_element_type=jnp.float32)
         + bg_ref[...])
    out_ref[...] = jnp.maximum(g, 0.0)


_MESH = plsc.VectorSubcoreMesh(core_axis_name="c", subcore_axis_name="s",
                               num_cores=NC, num_subcores=NS)


@functools.partial(
    pl.kernel,
    out_type=(jax.ShapeDtypeStruct((E * DE,), jnp.float32),
              jax.ShapeDtypeStruct((NW * N,), jnp.float32)),
    mesh=_MESH,
    scratch_types=[
        pltpu.VMEM((EPW,), jnp.int32),
        pltpu.VMEM((EPW,), jnp.int32),
        pltpu.VMEM((CH, D), jnp.float32),
        pltpu.VMEM((CH, D), jnp.float32),
        pltpu.VMEM((CH, D), jnp.float32),
        pltpu.VMEM((CH, D), jnp.float32),
        pltpu.VMEM((CH, D), jnp.float32),
        pltpu.VMEM((CH, D), jnp.float32),
        pltpu.VMEM((CH, DE), jnp.float32),
        pltpu.VMEM((CH, DE), jnp.float32),
        pltpu.VMEM((CH, DE), jnp.float32),
        pltpu.VMEM((CH * DE,), jnp.float32),
        pltpu.VMEM((CH * DE,), jnp.float32),
        pltpu.VMEM((N,), jnp.float32),
        pltpu.SemaphoreType.DMA,
        pltpu.SemaphoreType.DMA,
        pltpu.SemaphoreType.DMA,
        pltpu.SemaphoreType.DMA,
        pltpu.SemaphoreType.DMA,
    ],
    compiler_params=pltpu.CompilerParams(needs_layout_passes=False),
)
def _edge_sc(xsd_hbm, c_hbm, row_hbm, col_hbm, enew_hbm, pc_hbm,
             idx_ra, idx_ca, a0, a1, a2, b0, b1, b2_, c0, c1, c2_, o0, o1,
             cnt_v, si0, si1, si2, so0, so1):
    wid = lax.axis_index("c") * NS + lax.axis_index("s")
    base0 = wid * EPW
    A, B, C_ = (a0, a1, a2), (b0, b1, b2_), (c0, c1, c2_)
    O1 = (o0, o1)
    SI, SO = (si0, si1, si2), (so0, so1)

    def fill_zc(i, cc):
        cnt_v[pl.ds(16 * i, 16)] = jnp.zeros((16,), jnp.float32)
        return cc

    lax.fori_loop(0, N // 16, fill_zc, 0)

    pltpu.sync_copy(row_hbm.at[pl.ds(base0, EPW)], idx_ra)
    pltpu.sync_copy(col_hbm.at[pl.ds(base0, EPW)], idx_ca)

    def in_descs(k, s3):
        off = k * CH
        return (
            pltpu.make_async_copy(xsd_hbm.at[idx_ra.at[pl.ds(off, CH)]],
                                  A[s3], SI[s3]),
            pltpu.make_async_copy(xsd_hbm.at[idx_ca.at[pl.ds(off, CH)]],
                                  B[s3], SI[s3]),
            pltpu.make_async_copy(c_hbm.at[pl.ds(base0 + off, CH)],
                                  C_[s3], SI[s3]),
        )

    def out_desc(k, s2):
        return pltpu.make_async_copy(
            O1[s2], enew_hbm.at[pl.ds((base0 + k * CH) * DE, CH * DE)],
            SO[s2])

    def pipe_iter(k, s3, s2, in_loop):
        if in_loop:
            @pl.when(k >= 2)
            def _():
                out_desc(k - 2, s2).wait()
        elif k >= 2:
            out_desc(k - 2, s2).wait()

        if (not in_loop) and k + 2 > NIT - 1:
            pass
        else:
            for dsc in in_descs(k + 2, (s3 + 2) % 3):
                dsc.start()

        for dsc in in_descs(k, s3):
            dsc.wait()

        def body(i, cc):
            O1[s2][pl.ds(i * DE, DE)] = jnp.maximum(
                A[s3][i, pl.ds(0, 16)] + B[s3][i, pl.ds(16, 16)]
                + C_[s3][i], 0.0)
            return cc

        lax.fori_loop(0, CH, body, 0)
        out_desc(k, s2).start()

    for dsc in in_descs(0, 0):
        dsc.start()
    for dsc in in_descs(1, 1):
        dsc.start()

    NLOOP = (NIT - 5) // 6 * 6

    @pl.loop(0, NLOOP, step=6)
    def _(kk):
        for u in range(6):
            pipe_iter(kk + u, u % 3, u % 2, True)

    for k in range(NLOOP, NIT):
        pipe_iter(k, k % 3, k % 2, False)

    out_desc(NIT - 2, (NIT - 2) % 2).wait()
    out_desc(NIT - 1, (NIT - 1) % 2).wait()

    lanes = lax.iota(jnp.int32, 16)
    one16 = jnp.ones((16,), jnp.float32)

    def count(q, c2):
        idx16 = idx_ca[pl.ds(16 * q, 16)]
        for j in range(16):
            plsc.addupdate_scatter(cnt_v, [idx16], one16, mask=lanes == j)
        return c2

    lax.fori_loop(0, EPW // 16, count, 0)
    pltpu.sync_copy(cnt_v, pc_hbm.at[pl.ds(wid * N, N)])


@functools.partial(
    pl.kernel,
    out_type=jax.ShapeDtypeStruct((NC, N, D), jnp.float32),
    mesh=_MESH,
    scratch_types=[
        [pltpu.VMEM((CH4, D), jnp.float32)] * 4,
        [pltpu.VMEM((CH4, D), jnp.float32)] * 4,
        [pltpu.VMEM((CH4,), jnp.int32)] * 4,
        [pltpu.VMEM((CH4,), jnp.int32)] * 8,
        pltpu.VMEM_SHARED((N, D), jnp.float32),
        [pltpu.SemaphoreType.DMA] * 4,
        [pltpu.SemaphoreType.DMA] * 4,
        [pltpu.SemaphoreType.DMA] * 2,
    ],
    compiler_params=pltpu.CompilerParams(needs_layout_passes=False),
)
def _agg_sc(xm_hbm, t_hbm, row_hbm, col_hbm, pm_hbm,
            G, T_, IR, IC, sm, SI, SX, SS):
    cid = lax.axis_index("c")
    sid = lax.axis_index("s")
    wid = cid * NS + sid
    base0 = wid * EPW

    def fill_zb(i, c2):
        for j in range(D // 16):
            G[0][i, pl.ds(16 * j, 16)] = jnp.zeros((16,), jnp.float32)
        return c2

    lax.fori_loop(0, CH4, fill_zb, 0)
    for r in range(CPR // CH4):
        pltpu.sync_copy(G[0], sm.at[pl.ds(sid * CPR + r * CH4, CH4)])
    pltpu.sync_copy(G[0].at[pl.ds(0, CPR - (CPR // CH4) * CH4)],
                    sm.at[pl.ds(sid * CPR + (CPR // CH4) * CH4,
                                CPR - (CPR // CH4) * CH4)])

    @pl.when(sid == NS - 1)
    def _():
        pltpu.sync_copy(G[0].at[pl.ds(0, N - NS * CPR)],
                        sm.at[pl.ds(NS * CPR, N - NS * CPR)])

    plsc.subcore_barrier()

    def idx_descs(k, q4, q8):
        base = base0 + k * CH4
        return (
            pltpu.make_async_copy(row_hbm.at[pl.ds(base, CH4)], IR[q4],
                                  SX[q4]),
            pltpu.make_async_copy(col_hbm.at[pl.ds(base, CH4)], IC[q8],
                                  SX[q4]),
        )

    def gt_descs(k, q4):
        base = base0 + k * CH4
        return (
            pltpu.make_async_copy(xm_hbm.at[IR[q4]], G[q4], SI[q4]),
            pltpu.make_async_copy(t_hbm.at[pl.ds(base, CH4)], T_[q4],
                                  SI[q4]),
        )

    def scat_desc(q4, q8, s2):
        return pltpu.make_async_copy(G[q4], sm.at[IC[q8]], SS[s2])

    def pipe_iter(k, u, in_loop):
        q4 = u % 4
        q8 = u % 8
        for dsc in gt_descs(k, q4):
            dsc.wait()

        def body(i, c2):
            for j in range(D // 16):
                sl = pl.ds(16 * j, 16)
                G[q4][i, sl] = jnp.maximum(G[q4][i, sl] + T_[q4][i, sl], 0.0)
            return c2

        lax.fori_loop(0, CH4, body, 0)

        if in_loop:
            @pl.when(k >= 2)
            def _():
                scat_desc((u + 2) % 4, (u + 6) % 8, u % 2).wait()
        elif k >= 2:
            scat_desc((u + 2) % 4, (u + 6) % 8, u % 2).wait()

        pltpu.async_copy(G[q4], sm.at[IC[q8]], SS[u % 2], add=True)

        if in_loop:
            @pl.when(k + 4 <= NIT4 - 1)
            def _():
                for dsc in idx_descs(k + 4, q4, (u + 4) % 8):
                    dsc.start()
        elif k + 4 <= NIT4 - 1:
            for dsc in idx_descs(k + 4, q4, (u + 4) % 8):
                dsc.start()

        if in_loop or k + 2 <= NIT4 - 1:
            for dsc in idx_descs(k + 2, (u + 2) % 4, (u + 2) % 8):
                dsc.wait()
            for dsc in gt_descs(k + 2, (u + 2) % 4):
                dsc.start()

    for kp in range(4):
        for dsc in idx_descs(kp, kp % 4, kp % 8):
            dsc.start()
    for kp in range(2):
        for dsc in idx_descs(kp, kp % 4, kp % 8):
            dsc.wait()
        for dsc in gt_descs(kp, kp % 4):
            dsc.start()

    NLOOP4 = (NIT4 - 2) // 8 * 8

    @pl.loop(0, NLOOP4, step=8)
    def _(kk):
        for u in range(8):
            pipe_iter(kk + u, u, True)

    for k in range(NLOOP4, NIT4):
        pipe_iter(k, k % 8, False)

    scat_desc((NIT4 - 2) % 4, (NIT4 - 2) % 8, (NIT4 - 2) % 2).wait()
    scat_desc((NIT4 - 1) % 4, (NIT4 - 1) % 8, (NIT4 - 1) % 2).wait()

    plsc.subcore_barrier()

    pltpu.sync_copy(sm.at[pl.ds(sid * CPR, CPR)],
                    pm_hbm.at[cid, pl.ds(sid * CPR, CPR)])

    @pl.when(sid == NS - 1)
    def _():
        pltpu.sync_copy(sm.at[pl.ds(NS * CPR, N - NS * CPR)],
                        pm_hbm.at[cid, pl.ds(NS * CPR, N - NS * CPR)])


def kernel(x, edge_index, edge_attr, u, W_edge, b_edge, W_node1, b_node1,
           W_node2, b_node2, W_glob, b_glob):
    f32 = jnp.float32

    W_esrc = W_edge[:D]
    W_edst = W_edge[D:2 * D]
    W_ee = W_edge[2 * D:2 * D + DE]
    W_eu = W_edge[2 * D + DE:]
    W1x = W_node1[:D]
    W1e = W_node1[D:]
    W2x = W_node2[:D]
    W2a = W_node2[D:2 * D]
    W2u = W_node2[2 * D:]
    Wg_u = W_glob[:DU]
    Wg_x = W_glob[DU:]
    be2 = b_edge.reshape(1, DE)
    b12 = b_node1.reshape(1, D)
    b22 = b_node2.reshape(1, D)
    bg2 = b_glob.reshape(1, DU)

    NB = 5
    NBR = N // NB
    EB = 80
    EBR = E // EB

    Wsd = jnp.concatenate(
        [W_esrc, W_edst, jnp.zeros((D, D - 2 * DE), f32)], axis=1)
    xsd, xm = pl.pallas_call(
        _proj_body,
        grid=(NB,),
        in_specs=[
            pl.BlockSpec((NBR, D), lambda i: (i, 0)),
            pl.BlockSpec((D, D), lambda i: (0, 0)),
            pl.BlockSpec((D, D), lambda i: (0, 0)),
        ],
        out_specs=[
            pl.BlockSpec((NBR, D), lambda i: (i, 0)),
            pl.BlockSpec((NBR, D), lambda i: (i, 0)),
        ],
        out_shape=[
            jax.ShapeDtypeStruct((N, D), f32),
            jax.ShapeDtypeStruct((N, D), f32),
        ],
    )(x, Wsd, W1x)

    c = pl.pallas_call(
        _edgeconst_body,
        grid=(EB,),
        in_specs=[
            pl.BlockSpec((EBR, DE), lambda i: (i, 0)),
            pl.BlockSpec((DE, DE), lambda i: (0, 0)),
            pl.BlockSpec((1, DU), lambda i: (0, 0)),
            pl.BlockSpec((DU, DE), lambda i: (0, 0)),
            pl.BlockSpec((1, DE), lambda i: (0, 0)),
        ],
        out_specs=pl.BlockSpec((EBR, DE), lambda i: (i, 0)),
        out_shape=jax.ShapeDtypeStruct((E, DE), f32),
    )(edge_attr, W_ee, u, W_eu, be2)

    row = edge_index[0]
    col = edge_index[1]

    enew_flat, pc = _edge_sc(xsd, c, row, col)
    enew = enew_flat.reshape(E, DE)

    t = pl.pallas_call(
        _tmat_body,
        grid=(EB,),
        in_specs=[
            pl.BlockSpec((EBR, DE), lambda i: (i, 0)),
            pl.BlockSpec((DE, D), lambda i: (0, 0)),
            pl.BlockSpec((1, D), lambda i: (0, 0)),
        ],
        out_specs=pl.BlockSpec((EBR, D), lambda i: (i, 0)),
        out_shape=jax.ShapeDtypeStruct((E, D), f32),
    )(enew, W1e, b12)

    pm = _agg_sc(xm, t, row, col)

    pct = pc.reshape(NW, N).T
    new_x, s = pl.pallas_call(
        _node_body,
        grid=(NB,),
        in_specs=[
            pl.BlockSpec((NBR, D), lambda i: (i, 0)),
            pl.BlockSpec((NBR, D), lambda i: (i, 0)),
            pl.BlockSpec((NBR, D), lambda i: (i, 0)),
            pl.BlockSpec((NBR, NW), lambda i: (i, 0)),
            pl.BlockSpec((1, DU), lambda i: (0, 0)),
            pl.BlockSpec((D, D), lambda i: (0, 0)),
            pl.BlockSpec((D, D), lambda i: (0, 0)),
            pl.BlockSpec((DU, D), lambda i: (0, 0)),
            pl.BlockSpec((1, D), lambda i: (0, 0)),
        ],
        out_specs=[
            pl.BlockSpec((NBR, D), lambda i: (i, 0)),
            pl.BlockSpec((1, D), lambda i: (0, 0)),
        ],
        out_shape=[
            jax.ShapeDtypeStruct((N, D), f32),
            jax.ShapeDtypeStruct((1, D), f32),
        ],
    )(x, pm[0], pm[1], pct, u, W2x, W2a, W2u, b22)

    new_u = pl.pallas_call(
        _glob_body,
        in_specs=[
            pl.BlockSpec((1, DU), lambda: (0, 0)),
            pl.BlockSpec((1, D), lambda: (0, 0)),
            pl.BlockSpec((DU, DU), lambda: (0, 0)),
            pl.BlockSpec((D, DU), lambda: (0, 0)),
            pl.BlockSpec((1, DU), lambda: (0, 0)),
        ],
        out_specs=pl.BlockSpec((1, DU), lambda: (0, 0)),
        out_shape=jax.ShapeDtypeStruct((1, DU), f32),
    )(u, s, Wg_u, Wg_x, bg2)

    return (new_x, edge_index, enew, new_u)


# speedup vs baseline: 1.0673x; 1.0673x over previous
import functools

import jax
import jax.numpy as jnp
import numpy as np
from jax import lax
from jax.experimental import pallas as pl
from jax.experimental.pallas import tpu as pltpu
from jax.experimental.pallas import tpu_sc as plsc

N = 10000
E = 320000
D = 128
DE = 16
DU = 32

NC = 2
NS = 16
NW = NC * NS
EPW = E // NW
CH = 80
NIT = EPW // CH
CH4 = 40
NIT4 = EPW // CH4
CPR = 624

_TPERM = np.empty(D, np.int32)
for _j in range(D // 32):
    for _m in range(16):
        _TPERM[32 * _j + 2 * _m] = 32 * _j + _m
        _TPERM[32 * _j + 2 * _m + 1] = 32 * _j + 16 + _m
ZB = 52


def _proj_body(x_ref, wsd_ref, wm_ref, xsd_ref, xm_ref):
    xb = x_ref[...]
    xsd_ref[...] = jnp.dot(xb, wsd_ref[...], preferred_element_type=jnp.float32)
    xm_ref[...] = jnp.dot(xb, wm_ref[...], preferred_element_type=jnp.float32)


def _edgeconst_body(ea_ref, wee_ref, u_ref, weu_ref, be_ref, c_ref):
    cu = jnp.dot(u_ref[...], weu_ref[...], preferred_element_type=jnp.float32)
    c_ref[...] = (jnp.dot(ea_ref[...], wee_ref[...],
                          preferred_element_type=jnp.float32) + cu + be_ref[...])


def _tmat_body(e_ref, w_ref, b_ref, t_ref):
    t_ref[...] = jnp.dot(e_ref[...], w_ref[...],
                         preferred_element_type=jnp.float32) + b_ref[...]


def _node_body(x_ref, p0m_ref, p1m_ref, pc_ref, u_ref,
               w2x_ref, w2a_ref, w2u_ref, b2_ref, nx_ref, s_ref):
    i = pl.program_id(0)
    cnt = jnp.maximum(jnp.sum(pc_ref[...], axis=1, keepdims=True), 1.0)
    agg = (p0m_ref[...] + p1m_ref[...]) / cnt
    nx = jnp.dot(x_ref[...], w2x_ref[...], preferred_element_type=jnp.float32)
    nx = nx + jnp.dot(agg, w2a_ref[...], preferred_element_type=jnp.float32)
    nx = nx + jnp.dot(u_ref[...], w2u_ref[...], preferred_element_type=jnp.float32)
    nx = jnp.maximum(nx + b2_ref[...], 0.0)
    nx_ref[...] = nx

    @pl.when(i == 0)
    def _():
        s_ref[...] = jnp.zeros_like(s_ref)

    s_ref[...] += jnp.sum(nx, axis=0, keepdims=True)


def _glob_body(u_ref, s_ref, wgu_ref, wgx_ref, bg_ref, out_ref):
    m = s_ref[...] * (1.0 / N)
    g = (jnp.dot(u_ref[...], wgu_ref[...], preferred_element_type=jnp.float32)
         + jnp.dot(m, wgx_ref[...], preferred_element_type=jnp.float32)
         + bg_ref[...])
    out_ref[...] = jnp.maximum(g, 0.0)


_MESH = plsc.VectorSubcoreMesh(core_axis_name="c", subcore_axis_name="s",
                               num_cores=NC, num_subcores=NS)


@functools.partial(
    pl.kernel,
    out_type=(jax.ShapeDtypeStruct((E, DE), jnp.float32),
              jax.ShapeDtypeStruct((NW * N,), jnp.float32)),
    mesh=_MESH,
    scratch_types=[
        pltpu.VMEM((EPW,), jnp.int32),
        pltpu.VMEM((EPW,), jnp.int32),
        [pltpu.VMEM((CH, D), jnp.float32)] * 2,
        [pltpu.VMEM((CH, D), jnp.float32)] * 2,
        [pltpu.VMEM((CH, DE), jnp.float32)] * 2,
        [pltpu.VMEM((CH, DE), jnp.float32)] * 2,
        pltpu.VMEM((N,), jnp.float32),
        [pltpu.SemaphoreType.DMA] * 2,
        [pltpu.SemaphoreType.DMA] * 2,
    ],
    compiler_params=pltpu.CompilerParams(needs_layout_passes=False),
)
def _edge_sc(xsd_hbm, c_hbm, row_hbm, col_hbm, enew_hbm, pc_hbm,
             idx_ra, idx_ca, A, B, C_, O, cnt_v, SI, SO):
    wid = lax.axis_index("c") * NS + lax.axis_index("s")
    base0 = wid * EPW

    def fill_zc(i, c2):
        cnt_v[pl.ds(16 * i, 16)] = jnp.zeros((16,), jnp.float32)
        return c2

    lax.fori_loop(0, N // 16, fill_zc, 0)

    pltpu.sync_copy(row_hbm.at[pl.ds(base0, EPW)], idx_ra)
    pltpu.sync_copy(col_hbm.at[pl.ds(base0, EPW)], idx_ca)

    def in_descs(k, s):
        off = k * CH
        return (
            pltpu.make_async_copy(xsd_hbm.at[idx_ra.at[pl.ds(off, CH)]],
                                  A[s], SI[s]),
            pltpu.make_async_copy(xsd_hbm.at[idx_ca.at[pl.ds(off, CH)]],
                                  B[s], SI[s]),
            pltpu.make_async_copy(c_hbm.at[pl.ds(base0 + off, CH)],
                                  C_[s], SI[s]),
        )

    def out_desc(k, s):
        return pltpu.make_async_copy(
            O[s], enew_hbm.at[pl.ds(base0 + k * CH, CH)], SO[s])

    def process(k, s):
        for dsc in in_descs(k, s):
            dsc.wait()

        def body(i, c2):
            O[s][i] = jnp.maximum(
                A[s][i, pl.ds(0, 16)] + B[s][i, pl.ds(16, 16)] + C_[s][i], 0.0)
            return c2

        lax.fori_loop(0, CH, body, 0)
        out_desc(k, s).start()

    for dsc in in_descs(0, 0):
        dsc.start()

    @pl.loop(0, NIT - 1, step=2)
    def _(kk):
        for b2 in range(2):
            k = kk + b2
            s = b2
            for dsc in in_descs(k + 1, 1 - s):
                dsc.start()

            @pl.when(k >= 2)
            def _():
                out_desc(k, s).wait()

            process(k, s)

    out_desc(NIT - 1, 0).wait()
    process(NIT - 1, 0)
    out_desc(NIT - 2, 1).wait()
    out_desc(NIT - 1, 0).wait()

    lanes = lax.iota(jnp.int32, 16)
    one16 = jnp.ones((16,), jnp.float32)

    def count(q, c2):
        idx16 = idx_ca[pl.ds(16 * q, 16)]
        for j in range(16):
            plsc.addupdate_scatter(cnt_v, [idx16], one16, mask=lanes == j)
        return c2

    lax.fori_loop(0, EPW // 16, count, 0)
    pltpu.sync_copy(cnt_v, pc_hbm.at[pl.ds(wid * N, N)])


@functools.partial(
    pl.kernel,
    out_type=jax.ShapeDtypeStruct((NC, N, D), jnp.float32),
    mesh=_MESH,
    scratch_types=[
        [pltpu.VMEM((CH4, D), jnp.float32)] * 4,
        [pltpu.VMEM((2 * CH4, D), jnp.float32)] * 2,
        [pltpu.VMEM((CH4,), jnp.int32)] * 4,
        [pltpu.VMEM((CH4,), jnp.int32)] * 8,
        pltpu.VMEM_SHARED((N, D), jnp.float32),
        [pltpu.SemaphoreType.DMA] * 4,
        [pltpu.SemaphoreType.DMA] * 2,
        [pltpu.SemaphoreType.DMA] * 4,
        [pltpu.SemaphoreType.DMA] * 2,
    ],
    compiler_params=pltpu.CompilerParams(needs_layout_passes=False),
)
def _agg_sc(xm_hbm, t_hbm, row_hbm, col_hbm, pm_hbm,
            G, T2, IR, IC, sm, SI, ST, SX, SS):
    cid = lax.axis_index("c")
    sid = lax.axis_index("s")
    wid = cid * NS + sid
    base0 = wid * EPW

    def fill_zb(i, c2):
        for j in range(D // 16):
            G[0][i, pl.ds(16 * j, 16)] = jnp.zeros((16,), jnp.float32)
        return c2

    lax.fori_loop(0, CH4, fill_zb, 0)
    for r in range(CPR // CH4):
        pltpu.sync_copy(G[0], sm.at[pl.ds(sid * CPR + r * CH4, CH4)])
    pltpu.sync_copy(G[0].at[pl.ds(0, CPR - (CPR // CH4) * CH4)],
                    sm.at[pl.ds(sid * CPR + (CPR // CH4) * CH4,
                                CPR - (CPR // CH4) * CH4)])

    @pl.when(sid == NS - 1)
    def _():
        pltpu.sync_copy(G[0].at[pl.ds(0, N - NS * CPR)],
                        sm.at[pl.ds(NS * CPR, N - NS * CPR)])

    plsc.subcore_barrier()

    def idx_descs(k, q4, q8):
        base = base0 + k * CH4
        return (
            pltpu.make_async_copy(row_hbm.at[pl.ds(base, CH4)], IR[q4],
                                  SX[q4]),
            pltpu.make_async_copy(col_hbm.at[pl.ds(base, CH4)], IC[q8],
                                  SX[q4]),
        )

    def g_desc(k, q4):
        return pltpu.make_async_copy(xm_hbm.at[IR[q4]], G[q4], SI[q4])

    def t_desc(p, tslot):
        return pltpu.make_async_copy(
            t_hbm.at[pl.ds(base0 + p * 2 * CH4, 2 * CH4)], T2[tslot],
            ST[tslot])

    def scat_desc(q4, q8, s2):
        return pltpu.make_async_copy(G[q4], sm.at[IC[q8]], SS[s2])

    def pipe_iter(k, u, in_loop):
        q4 = u % 4
        q8 = u % 8
        tslot = (u // 2) % 2
        half = (u % 2) * CH4
        g_desc(k, q4).wait()
        if u % 2 == 0:
            t_desc(k // 2, tslot).wait()

        def body(i, c2):
            for j in range(D // 16):
                sl = pl.ds(16 * j, 16)
                G[q4][i, sl] = jnp.maximum(
                    G[q4][i, sl] + T2[tslot][half + i, sl], 0.0)
            return c2

        lax.fori_loop(0, CH4, body, 0)

        if in_loop:
            @pl.when(k >= 2)
            def _():
                scat_desc((u + 2) % 4, (u + 6) % 8, u % 2).wait()
        elif k >= 2:
            scat_desc((u + 2) % 4, (u + 6) % 8, u % 2).wait()

        pltpu.async_copy(G[q4], sm.at[IC[q8]], SS[u % 2], add=True)

        if in_loop:
            @pl.when(k + 4 <= NIT4 - 1)
            def _():
                for dsc in idx_descs(k + 4, q4, (u + 4) % 8):
                    dsc.start()
        elif k + 4 <= NIT4 - 1:
            for dsc in idx_descs(k + 4, q4, (u + 4) % 8):
                dsc.start()

        if in_loop or k + 2 <= NIT4 - 1:
            for dsc in idx_descs(k + 2, (u + 2) % 4, (u + 2) % 8):
                dsc.wait()
            g_desc(k + 2, (u + 2) % 4).start()
            if u % 2 == 0:
                t_desc(k // 2 + 1, (tslot + 1) % 2).start()

    for kp in range(4):
        for dsc in idx_descs(kp, kp % 4, kp % 8):
            dsc.start()
    for kp in range(2):
        for dsc in idx_descs(kp, kp % 4, kp % 8):
            dsc.wait()
        g_desc(kp, kp % 4).start()
    t_desc(0, 0).start()

    NLOOP4 = (NIT4 - 2) // 8 * 8

    @pl.loop(0, NLOOP4, step=8)
    def _(kk):
        for u in range(8):
            pipe_iter(kk + u, u, True)

    for k in range(NLOOP4, NIT4):
        pipe_iter(k, k % 8, False)

    scat_desc((NIT4 - 2) % 4, (NIT4 - 2) % 8, (NIT4 - 2) % 2).wait()
    scat_desc((NIT4 - 1) % 4, (NIT4 - 1) % 8, (NIT4 - 1) % 2).wait()

    plsc.subcore_barrier()

    pltpu.sync_copy(sm.at[pl.ds(sid * CPR, CPR)],
                    pm_hbm.at[cid, pl.ds(sid * CPR, CPR)])

    @pl.when(sid == NS - 1)
    def _():
        pltpu.sync_copy(sm.at[pl.ds(NS * CPR, N - NS * CPR)],
                        pm_hbm.at[cid, pl.ds(NS * CPR, N - NS * CPR)])


def kernel(x, edge_index, edge_attr, u, W_edge, b_edge, W_node1, b_node1,
           W_node2, b_node2, W_glob, b_glob):
    f32 = jnp.float32

    W_esrc = W_edge[:D]
    W_edst = W_edge[D:2 * D]
    W_ee = W_edge[2 * D:2 * D + DE]
    W_eu = W_edge[2 * D + DE:]
    W1x = W_node1[:D]
    W1e = W_node1[D:]
    W2x = W_node2[:D]
    W2a = W_node2[D:2 * D]
    W2u = W_node2[2 * D:]
    Wg_u = W_glob[:DU]
    Wg_x = W_glob[DU:]
    be2 = b_edge.reshape(1, DE)
    b12 = b_node1.reshape(1, D)
    b22 = b_node2.reshape(1, D)
    bg2 = b_glob.reshape(1, DU)

    NB = 5
    NBR = N // NB
    EB = 80
    EBR = E // EB

    Wsd = jnp.concatenate(
        [W_esrc, W_edst, jnp.zeros((D, D - 2 * DE), f32)], axis=1)
    xsd, xm = pl.pallas_call(
        _proj_body,
        grid=(NB,),
        in_specs=[
            pl.BlockSpec((NBR, D), lambda i: (i, 0)),
            pl.BlockSpec((D, D), lambda i: (0, 0)),
            pl.BlockSpec((D, D), lambda i: (0, 0)),
        ],
        out_specs=[
            pl.BlockSpec((NBR, D), lambda i: (i, 0)),
            pl.BlockSpec((NBR, D), lambda i: (i, 0)),
        ],
        out_shape=[
            jax.ShapeDtypeStruct((N, D), f32),
            jax.ShapeDtypeStruct((N, D), f32),
        ],
    )(x, Wsd, W1x)

    c = pl.pallas_call(
        _edgeconst_body,
        grid=(EB,),
        in_specs=[
            pl.BlockSpec((EBR, DE), lambda i: (i, 0)),
            pl.BlockSpec((DE, DE), lambda i: (0, 0)),
            pl.BlockSpec((1, DU), lambda i: (0, 0)),
            pl.BlockSpec((DU, DE), lambda i: (0, 0)),
            pl.BlockSpec((1, DE), lambda i: (0, 0)),
        ],
        out_specs=pl.BlockSpec((EBR, DE), lambda i: (i, 0)),
        out_shape=jax.ShapeDtypeStruct((E, DE), f32),
    )(edge_attr, W_ee, u, W_eu, be2)

    row = edge_index[0]
    col = edge_index[1]

    enew, pc = _edge_sc(xsd, c, row, col)

    t = pl.pallas_call(
        _tmat_body,
        grid=(EB,),
        in_specs=[
            pl.BlockSpec((EBR, DE), lambda i: (i, 0)),
            pl.BlockSpec((DE, D), lambda i: (0, 0)),
            pl.BlockSpec((1, D), lambda i: (0, 0)),
        ],
        out_specs=pl.BlockSpec((EBR, D), lambda i: (i, 0)),
        out_shape=jax.ShapeDtypeStruct((E, D), f32),
    )(enew, W1e, b12)

    pm = _agg_sc(xm, t, row, col)

    pct = pc.reshape(NW, N).T
    new_x, s = pl.pallas_call(
        _node_body,
        grid=(NB,),
        in_specs=[
            pl.BlockSpec((NBR, D), lambda i: (i, 0)),
            pl.BlockSpec((NBR, D), lambda i: (i, 0)),
            pl.BlockSpec((NBR, D), lambda i: (i, 0)),
            pl.BlockSpec((NBR, NW), lambda i: (i, 0)),
            pl.BlockSpec((1, DU), lambda i: (0, 0)),
            pl.BlockSpec((D, D), lambda i: (0, 0)),
            pl.BlockSpec((D, D), lambda i: (0, 0)),
            pl.BlockSpec((DU, D), lambda i: (0, 0)),
            pl.BlockSpec((1, D), lambda i: (0, 0)),
        ],
        out_specs=[
            pl.BlockSpec((NBR, D), lambda i: (i, 0)),
            pl.BlockSpec((1, D), lambda i: (0, 0)),
        ],
        out_shape=[
            jax.ShapeDtypeStruct((N, D), f32),
            jax.ShapeDtypeStruct((1, D), f32),
        ],
    )(x, pm[0], pm[1], pct, u, W2x, W2a, W2u, b22)

    new_u = pl.pallas_call(
        _glob_body,
        in_specs=[
            pl.BlockSpec((1, DU), lambda: (0, 0)),
            pl.BlockSpec((1, D), lambda: (0, 0)),
            pl.BlockSpec((DU, DU), lambda: (0, 0)),
            pl.BlockSpec((D, DU), lambda: (0, 0)),
            pl.BlockSpec((1, DU), lambda: (0, 0)),
        ],
        out_specs=pl.BlockSpec((1, DU), lambda: (0, 0)),
        out_shape=jax.ShapeDtypeStruct((1, DU), f32),
    )(u, s, Wg_u, Wg_x, bg2)

    return (new_x, edge_index, enew, new_u)


# speedup vs baseline: 1.0833x; 1.0149x over previous
import functools

import jax
import jax.numpy as jnp
import numpy as np
from jax import lax
from jax.experimental import pallas as pl
from jax.experimental.pallas import tpu as pltpu
from jax.experimental.pallas import tpu_sc as plsc

N = 10000
E = 320000
D = 128
DE = 16
DU = 32

NC = 2
NS = 16
NW = NC * NS
EPW = E // NW
CH = 80
NIT = EPW // CH
CH4 = 40
NIT4 = EPW // CH4
CPR = 624

_TPERM = np.empty(D, np.int32)
for _j in range(D // 32):
    for _m in range(16):
        _TPERM[32 * _j + 2 * _m] = 32 * _j + _m
        _TPERM[32 * _j + 2 * _m + 1] = 32 * _j + 16 + _m
ZB = 52


def _proj_body(x_ref, wsd_ref, wm_ref, xsd_ref, xm_ref):
    xb = x_ref[...]
    xsd_ref[...] = jnp.dot(xb, wsd_ref[...], preferred_element_type=jnp.float32)
    xm_ref[...] = jnp.dot(xb, wm_ref[...], preferred_element_type=jnp.float32)


def _edgeconst_body(ea_ref, wee_ref, u_ref, weu_ref, be_ref, c_ref):
    cu = jnp.dot(u_ref[...], weu_ref[...], preferred_element_type=jnp.float32)
    c_ref[...] = (jnp.dot(ea_ref[...], wee_ref[...],
                          preferred_element_type=jnp.float32) + cu + be_ref[...])


def _tmat_body(e_ref, w_ref, b_ref, t_ref):
    t_ref[...] = jnp.dot(e_ref[...], w_ref[...],
                         preferred_element_type=jnp.float32) + b_ref[...]


def _node_body(x_ref, p0m_ref, p1m_ref, pc_ref, u_ref,
               w2x_ref, w2a_ref, w2u_ref, b2_ref,
               wgu_ref, wgx_ref, bg_ref, nx_ref, nu_ref, s_ref):
    i = pl.program_id(0)
    cnt = jnp.maximum(jnp.sum(pc_ref[...], axis=1, keepdims=True), 1.0)
    agg = (p0m_ref[...] + p1m_ref[...]) / cnt
    nx = jnp.dot(x_ref[...], w2x_ref[...], preferred_element_type=jnp.float32)
    nx = nx + jnp.dot(agg, w2a_ref[...], preferred_element_type=jnp.float32)
    nx = nx + jnp.dot(u_ref[...], w2u_ref[...], preferred_element_type=jnp.float32)
    nx = jnp.maximum(nx + b2_ref[...], 0.0)
    nx_ref[...] = nx

    @pl.when(i == 0)
    def _():
        s_ref[...] = jnp.zeros_like(s_ref)

    s_ref[...] += jnp.sum(nx, axis=0, keepdims=True)

    @pl.when(i == pl.num_programs(0) - 1)
    def _():
        m = s_ref[...] * (1.0 / N)
        g = (jnp.dot(u_ref[...], wgu_ref[...],
                     preferred_element_type=jnp.float32)
             + jnp.dot(m, wgx_ref[...], preferred_element_type=jnp.float32)
             + bg_ref[...])
        nu_ref[...] = jnp.maximum(g, 0.0)


_MESH = plsc.VectorSubcoreMesh(core_axis_name="c", subcore_axis_name="s",
                               num_cores=NC, num_subcores=NS)


@functools.partial(
    pl.kernel,
    out_type=(jax.ShapeDtypeStruct((E, DE), jnp.float32),
              jax.ShapeDtypeStruct((NW * N,), jnp.float32)),
    mesh=_MESH,
    scratch_types=[
        pltpu.VMEM((EPW,), jnp.int32),
        pltpu.VMEM((EPW,), jnp.int32),
        [pltpu.VMEM((CH, D), jnp.float32)] * 2,
        [pltpu.VMEM((CH, D), jnp.float32)] * 2,
        [pltpu.VMEM((CH, DE), jnp.float32)] * 2,
        [pltpu.VMEM((CH, DE), jnp.float32)] * 2,
        pltpu.VMEM((N,), jnp.float32),
        [pltpu.SemaphoreType.DMA] * 2,
        [pltpu.SemaphoreType.DMA] * 2,
    ],
    compiler_params=pltpu.CompilerParams(needs_layout_passes=False),
)
def _edge_sc(xsd_hbm, c_hbm, row_hbm, col_hbm, enew_hbm, pc_hbm,
             idx_ra, idx_ca, A, B, C_, O, cnt_v, SI, SO):
    wid = lax.axis_index("c") * NS + lax.axis_index("s")
    base0 = wid * EPW

    def fill_zc(i, c2):
        cnt_v[pl.ds(16 * i, 16)] = jnp.zeros((16,), jnp.float32)
        return c2

    lax.fori_loop(0, N // 16, fill_zc, 0)

    pltpu.sync_copy(row_hbm.at[pl.ds(base0, EPW)], idx_ra)
    pltpu.sync_copy(col_hbm.at[pl.ds(base0, EPW)], idx_ca)

    def in_descs(k, s):
        off = k * CH
        return (
            pltpu.make_async_copy(xsd_hbm.at[idx_ra.at[pl.ds(off, CH)]],
                                  A[s], SI[s]),
            pltpu.make_async_copy(xsd_hbm.at[idx_ca.at[pl.ds(off, CH)]],
                                  B[s], SI[s]),
            pltpu.make_async_copy(c_hbm.at[pl.ds(base0 + off, CH)],
                                  C_[s], SI[s]),
        )

    def out_desc(k, s):
        return pltpu.make_async_copy(
            O[s], enew_hbm.at[pl.ds(base0 + k * CH, CH)], SO[s])

    def process(k, s):
        for dsc in in_descs(k, s):
            dsc.wait()

        def body(i, c2):
            O[s][i] = jnp.maximum(
                A[s][i, pl.ds(0, 16)] + B[s][i, pl.ds(16, 16)] + C_[s][i], 0.0)
            return c2

        lax.fori_loop(0, CH, body, 0)
        out_desc(k, s).start()

    for dsc in in_descs(0, 0):
        dsc.start()

    @pl.loop(0, NIT - 1, step=2)
    def _(kk):
        for b2 in range(2):
            k = kk + b2
            s = b2
            for dsc in in_descs(k + 1, 1 - s):
                dsc.start()

            @pl.when(k >= 2)
            def _():
                out_desc(k, s).wait()

            process(k, s)

    out_desc(NIT - 1, 0).wait()
    process(NIT - 1, 0)
    out_desc(NIT - 2, 1).wait()
    out_desc(NIT - 1, 0).wait()

    lanes = lax.iota(jnp.int32, 16)
    one16 = jnp.ones((16,), jnp.float32)

    def count(q, c2):
        idx16 = idx_ca[pl.ds(16 * q, 16)]
        for j in range(16):
            plsc.addupdate_scatter(cnt_v, [idx16], one16, mask=lanes == j)
        return c2

    lax.fori_loop(0, EPW // 16, count, 0)
    pltpu.sync_copy(cnt_v, pc_hbm.at[pl.ds(wid * N, N)])


@functools.partial(
    pl.kernel,
    out_type=jax.ShapeDtypeStruct((NC, N, D), jnp.float32),
    mesh=_MESH,
    scratch_types=[
        [pltpu.VMEM((CH4, D), jnp.float32)] * 4,
        [pltpu.VMEM((CH4, D), jnp.float32)] * 4,
        [pltpu.VMEM((CH4,), jnp.int32)] * 4,
        [pltpu.VMEM((CH4,), jnp.int32)] * 8,
        pltpu.VMEM_SHARED((N, D), jnp.float32),
        [pltpu.SemaphoreType.DMA] * 4,
        [pltpu.SemaphoreType.DMA] * 4,
        [pltpu.SemaphoreType.DMA] * 2,
    ],
    compiler_params=pltpu.CompilerParams(needs_layout_passes=False),
)
def _agg_sc(xm_hbm, t_hbm, row_hbm, col_hbm, pm_hbm,
            G, T_, IR, IC, sm, SI, SX, SS):
    cid = lax.axis_index("c")
    sid = lax.axis_index("s")
    wid = cid * NS + sid
    base0 = wid * EPW

    def fill_zb(i, c2):
        for j in range(D // 16):
            G[0][i, pl.ds(16 * j, 16)] = jnp.zeros((16,), jnp.float32)
        return c2

    lax.fori_loop(0, CH4, fill_zb, 0)
    for r in range(CPR // CH4):
        pltpu.sync_copy(G[0], sm.at[pl.ds(sid * CPR + r * CH4, CH4)])
    pltpu.sync_copy(G[0].at[pl.ds(0, CPR - (CPR // CH4) * CH4)],
                    sm.at[pl.ds(sid * CPR + (CPR // CH4) * CH4,
                                CPR - (CPR // CH4) * CH4)])

    @pl.when(sid == NS - 1)
    def _():
        pltpu.sync_copy(G[0].at[pl.ds(0, N - NS * CPR)],
                        sm.at[pl.ds(NS * CPR, N - NS * CPR)])

    plsc.subcore_barrier()

    def idx_descs(k, q4, q8):
        base = base0 + k * CH4
        return (
            pltpu.make_async_copy(row_hbm.at[pl.ds(base, CH4)], IR[q4],
                                  SX[q4]),
            pltpu.make_async_copy(col_hbm.at[pl.ds(base, CH4)], IC[q8],
                                  SX[q4]),
        )

    def gt_descs(k, q4):
        base = base0 + k * CH4
        return (
            pltpu.make_async_copy(xm_hbm.at[IR[q4]], G[q4], SI[q4]),
            pltpu.make_async_copy(t_hbm.at[pl.ds(base, CH4)], T_[q4],
                                  SI[q4]),
        )

    def scat_desc(q4, q8, s2):
        return pltpu.make_async_copy(G[q4], sm.at[IC[q8]], SS[s2])

    def pipe_iter(k, u, in_loop):
        q4 = u % 4
        q8 = u % 8
        for dsc in gt_descs(k, q4):
            dsc.wait()

        def body(i, c2):
            for j in range(D // 16):
                sl = pl.ds(16 * j, 16)
                G[q4][i, sl] = jnp.maximum(G[q4][i, sl] + T_[q4][i, sl], 0.0)
            return c2

        lax.fori_loop(0, CH4, body, 0)

        if in_loop:
            @pl.when(k >= 2)
            def _():
                scat_desc((u + 2) % 4, (u + 6) % 8, u % 2).wait()
        elif k >= 2:
            scat_desc((u + 2) % 4, (u + 6) % 8, u % 2).wait()

        pltpu.async_copy(G[q4], sm.at[IC[q8]], SS[u % 2], add=True)

        if in_loop:
            @pl.when(k + 4 <= NIT4 - 1)
            def _():
                for dsc in idx_descs(k + 4, q4, (u + 4) % 8):
                    dsc.start()
        elif k + 4 <= NIT4 - 1:
            for dsc in idx_descs(k + 4, q4, (u + 4) % 8):
                dsc.start()

        if in_loop or k + 2 <= NIT4 - 1:
            for dsc in idx_descs(k + 2, (u + 2) % 4, (u + 2) % 8):
                dsc.wait()
            for dsc in gt_descs(k + 2, (u + 2) % 4):
                dsc.start()

    for kp in range(4):
        for dsc in idx_descs(kp, kp % 4, kp % 8):
            dsc.start()
    for kp in range(2):
        for dsc in idx_descs(kp, kp % 4, kp % 8):
            dsc.wait()
        for dsc in gt_descs(kp, kp % 4):
            dsc.start()

    NLOOP4 = (NIT4 - 2) // 8 * 8

    @pl.loop(0, NLOOP4, step=8)
    def _(kk):
        for u in range(8):
            pipe_iter(kk + u, u, True)

    for k in range(NLOOP4, NIT4):
        pipe_iter(k, k % 8, False)

    scat_desc((NIT4 - 2) % 4, (NIT4 - 2) % 8, (NIT4 - 2) % 2).wait()
    scat_desc((NIT4 - 1) % 4, (NIT4 - 1) % 8, (NIT4 - 1) % 2).wait()

    plsc.subcore_barrier()

    pltpu.sync_copy(sm.at[pl.ds(sid * CPR, CPR)],
                    pm_hbm.at[cid, pl.ds(sid * CPR, CPR)])

    @pl.when(sid == NS - 1)
    def _():
        pltpu.sync_copy(sm.at[pl.ds(NS * CPR, N - NS * CPR)],
                        pm_hbm.at[cid, pl.ds(NS * CPR, N - NS * CPR)])


def kernel(x, edge_index, edge_attr, u, W_edge, b_edge, W_node1, b_node1,
           W_node2, b_node2, W_glob, b_glob):
    f32 = jnp.float32

    W_esrc = W_edge[:D]
    W_edst = W_edge[D:2 * D]
    W_ee = W_edge[2 * D:2 * D + DE]
    W_eu = W_edge[2 * D + DE:]
    W1x = W_node1[:D]
    W1e = W_node1[D:]
    W2x = W_node2[:D]
    W2a = W_node2[D:2 * D]
    W2u = W_node2[2 * D:]
    Wg_u = W_glob[:DU]
    Wg_x = W_glob[DU:]
    be2 = b_edge.reshape(1, DE)
    b12 = b_node1.reshape(1, D)
    b22 = b_node2.reshape(1, D)
    bg2 = b_glob.reshape(1, DU)

    NB = 5
    NBR = N // NB
    EB = 80
    EBR = E // EB

    Wsd = jnp.concatenate(
        [W_esrc, W_edst, jnp.zeros((D, D - 2 * DE), f32)], axis=1)
    xsd, xm = pl.pallas_call(
        _proj_body,
        grid=(NB,),
        in_specs=[
            pl.BlockSpec((NBR, D), lambda i: (i, 0)),
            pl.BlockSpec((D, D), lambda i: (0, 0)),
            pl.BlockSpec((D, D), lambda i: (0, 0)),
        ],
        out_specs=[
            pl.BlockSpec((NBR, D), lambda i: (i, 0)),
            pl.BlockSpec((NBR, D), lambda i: (i, 0)),
        ],
        out_shape=[
            jax.ShapeDtypeStruct((N, D), f32),
            jax.ShapeDtypeStruct((N, D), f32),
        ],
    )(x, Wsd, W1x)

    c = pl.pallas_call(
        _edgeconst_body,
        grid=(EB,),
        in_specs=[
            pl.BlockSpec((EBR, DE), lambda i: (i, 0)),
            pl.BlockSpec((DE, DE), lambda i: (0, 0)),
            pl.BlockSpec((1, DU), lambda i: (0, 0)),
            pl.BlockSpec((DU, DE), lambda i: (0, 0)),
            pl.BlockSpec((1, DE), lambda i: (0, 0)),
        ],
        out_specs=pl.BlockSpec((EBR, DE), lambda i: (i, 0)),
        out_shape=jax.ShapeDtypeStruct((E, DE), f32),
    )(edge_attr, W_ee, u, W_eu, be2)

    row = edge_index[0]
    col = edge_index[1]

    enew, pc = _edge_sc(xsd, c, row, col)

    t = pl.pallas_call(
        _tmat_body,
        grid=(EB,),
        in_specs=[
            pl.BlockSpec((EBR, DE), lambda i: (i, 0)),
            pl.BlockSpec((DE, D), lambda i: (0, 0)),
            pl.BlockSpec((1, D), lambda i: (0, 0)),
        ],
        out_specs=pl.BlockSpec((EBR, D), lambda i: (i, 0)),
        out_shape=jax.ShapeDtypeStruct((E, D), f32),
    )(enew, W1e, b12)

    pm = _agg_sc(xm, t, row, col)

    pct = pc.reshape(NW, N).T
    new_x, new_u = pl.pallas_call(
        _node_body,
        grid=(NB,),
        in_specs=[
            pl.BlockSpec((NBR, D), lambda i: (i, 0)),
            pl.BlockSpec((NBR, D), lambda i: (i, 0)),
            pl.BlockSpec((NBR, D), lambda i: (i, 0)),
            pl.BlockSpec((NBR, NW), lambda i: (i, 0)),
            pl.BlockSpec((1, DU), lambda i: (0, 0)),
            pl.BlockSpec((D, D), lambda i: (0, 0)),
            pl.BlockSpec((D, D), lambda i: (0, 0)),
            pl.BlockSpec((DU, D), lambda i: (0, 0)),
            pl.BlockSpec((1, D), lambda i: (0, 0)),
            pl.BlockSpec((DU, DU), lambda i: (0, 0)),
            pl.BlockSpec((D, DU), lambda i: (0, 0)),
            pl.BlockSpec((1, DU), lambda i: (0, 0)),
        ],
        out_specs=[
            pl.BlockSpec((NBR, D), lambda i: (i, 0)),
            pl.BlockSpec((1, DU), lambda i: (0, 0)),
        ],
        out_shape=[
            jax.ShapeDtypeStruct((N, D), f32),
            jax.ShapeDtypeStruct((1, DU), f32),
        ],
        scratch_shapes=[pltpu.VMEM((1, D), f32)],
    )(x, pm[0], pm[1], pct, u, W2x, W2a, W2u, b22, Wg_u, Wg_x, bg2)

    return (new_x, edge_index, enew, new_u)


# speedup vs baseline: 1.1070x; 1.0219x over previous
import functools

import jax
import jax.numpy as jnp
import numpy as np
from jax import lax
from jax.experimental import pallas as pl
from jax.experimental.pallas import tpu as pltpu
from jax.experimental.pallas import tpu_sc as plsc

N = 10000
E = 320000
D = 128
DE = 16
DU = 32

NC = 2
NS = 16
NW = NC * NS
EPW = E // NW
CH = 80
NIT = EPW // CH
CH4 = 40
NIT4 = EPW // CH4
CPR = 624

_TPERM = np.empty(D, np.int32)
for _j in range(D // 32):
    for _m in range(16):
        _TPERM[32 * _j + 2 * _m] = 32 * _j + _m
        _TPERM[32 * _j + 2 * _m + 1] = 32 * _j + 16 + _m
ZB = 52


def _proj_body(x_ref, wsd_ref, wm_ref, xsd_ref, xm_ref):
    xb = x_ref[...]
    xsd_ref[...] = jnp.dot(xb, wsd_ref[...], preferred_element_type=jnp.float32)
    xm_ref[...] = jnp.dot(xb, wm_ref[...], preferred_element_type=jnp.float32)


def _edgeconst_body(ea_ref, wee_ref, u_ref, weu_ref, be_ref, c_ref):
    cu = jnp.dot(u_ref[...], weu_ref[...], preferred_element_type=jnp.float32)
    c_ref[...] = (jnp.dot(ea_ref[...], wee_ref[...],
                          preferred_element_type=jnp.float32) + cu + be_ref[...])


def _tmat_body(e_ref, w_ref, b_ref, t_ref):
    t_ref[...] = jnp.dot(e_ref[...], w_ref[...],
                         preferred_element_type=jnp.float32) + b_ref[...]


def _node_body(x_ref, p0m_ref, p1m_ref, pc_ref, u_ref,
               w2x_ref, w2a_ref, w2u_ref, b2_ref,
               wgu_ref, wgx_ref, bg_ref, nx_ref, nu_ref, s_ref):
    i = pl.program_id(0)
    cnt = jnp.maximum(jnp.sum(pc_ref[...], axis=1, keepdims=True), 1.0)
    agg = (p0m_ref[...] + p1m_ref[...]) / cnt
    nx = jnp.dot(x_ref[...], w2x_ref[...], preferred_element_type=jnp.float32)
    nx = nx + jnp.dot(agg, w2a_ref[...], preferred_element_type=jnp.float32)
    nx = nx + jnp.dot(u_ref[...], w2u_ref[...], preferred_element_type=jnp.float32)
    nx = jnp.maximum(nx + b2_ref[...], 0.0)
    nx_ref[...] = nx

    @pl.when(i == 0)
    def _():
        s_ref[...] = jnp.zeros_like(s_ref)

    s_ref[...] += jnp.sum(nx, axis=0, keepdims=True)

    @pl.when(i == pl.num_programs(0) - 1)
    def _():
        m = s_ref[...] * (1.0 / N)
        g = (jnp.dot(u_ref[...], wgu_ref[...],
                     preferred_element_type=jnp.float32)
             + jnp.dot(m, wgx_ref[...], preferred_element_type=jnp.float32)
             + bg_ref[...])
        nu_ref[...] = jnp.maximum(g, 0.0)


_MESH = plsc.VectorSubcoreMesh(core_axis_name="c", subcore_axis_name="s",
                               num_cores=NC, num_subcores=NS)


@functools.partial(
    pl.kernel,
    out_type=(jax.ShapeDtypeStruct((E, DE), jnp.float32),
              jax.ShapeDtypeStruct((NW * N,), jnp.float32)),
    mesh=_MESH,
    scratch_types=[
        pltpu.VMEM((EPW,), jnp.int32),
        pltpu.VMEM((EPW,), jnp.int32),
        [pltpu.VMEM((CH, D), jnp.float32)] * 2,
        [pltpu.VMEM((CH, D), jnp.float32)] * 2,
        [pltpu.VMEM((CH, DE), jnp.float32)] * 2,
        [pltpu.VMEM((CH, DE), jnp.float32)] * 2,
        pltpu.VMEM((N,), jnp.float32),
        [pltpu.SemaphoreType.DMA] * 2,
        [pltpu.SemaphoreType.DMA] * 2,
    ],
    compiler_params=pltpu.CompilerParams(needs_layout_passes=False),
)
def _edge_sc(xsd_hbm, c_hbm, row_hbm, col_hbm, enew_hbm, pc_hbm,
             idx_ra, idx_ca, A, B, C_, O, cnt_v, SI, SO):
    wid = lax.axis_index("c") * NS + lax.axis_index("s")
    base0 = wid * EPW

    def fill_zc(i, c2):
        cnt_v[pl.ds(16 * i, 16)] = jnp.zeros((16,), jnp.float32)
        return c2

    lax.fori_loop(0, N // 16, fill_zc, 0)

    pltpu.sync_copy(row_hbm.at[pl.ds(base0, EPW)], idx_ra)
    pltpu.sync_copy(col_hbm.at[pl.ds(base0, EPW)], idx_ca)

    def in_descs(k, s):
        off = k * CH
        return (
            pltpu.make_async_copy(xsd_hbm.at[idx_ra.at[pl.ds(off, CH)]],
                                  A[s], SI[s]),
            pltpu.make_async_copy(xsd_hbm.at[idx_ca.at[pl.ds(off, CH)]],
                                  B[s], SI[s]),
            pltpu.make_async_copy(c_hbm.at[pl.ds(base0 + off, CH)],
                                  C_[s], SI[s]),
        )

    def out_desc(k, s):
        return pltpu.make_async_copy(
            O[s], enew_hbm.at[pl.ds(base0 + k * CH, CH)], SO[s])

    def process(k, s):
        for dsc in in_descs(k, s):
            dsc.wait()

        def body(i, c2):
            O[s][i] = jnp.maximum(
                A[s][i, pl.ds(0, 16)] + B[s][i, pl.ds(16, 16)] + C_[s][i], 0.0)
            return c2

        lax.fori_loop(0, CH, body, 0)
        out_desc(k, s).start()

    for dsc in in_descs(0, 0):
        dsc.start()

    @pl.loop(0, NIT - 1, step=2)
    def _(kk):
        for b2 in range(2):
            k = kk + b2
            s = b2
            for dsc in in_descs(k + 1, 1 - s):
                dsc.start()

            @pl.when(k >= 2)
            def _():
                out_desc(k, s).wait()

            process(k, s)

    out_desc(NIT - 1, 0).wait()
    process(NIT - 1, 0)
    out_desc(NIT - 2, 1).wait()
    out_desc(NIT - 1, 0).wait()

    lanes = lax.iota(jnp.int32, 16)
    one16 = jnp.ones((16,), jnp.float32)

    def count(q, c2):
        idx16 = idx_ca[pl.ds(16 * q, 16)]
        for j in range(16):
            plsc.addupdate_scatter(cnt_v, [idx16], one16, mask=lanes == j)
        return c2

    lax.fori_loop(0, EPW // 16, count, 0)
    pltpu.sync_copy(cnt_v, pc_hbm.at[pl.ds(wid * N, N)])


@functools.partial(
    pl.kernel,
    out_type=jax.ShapeDtypeStruct((NC, N, D), jnp.float32),
    mesh=_MESH,
    scratch_types=[
        [pltpu.VMEM((CH, D), jnp.float32)] * 2,
        [pltpu.VMEM((CH, D), jnp.float32)] * 2,
        [pltpu.VMEM((CH,), jnp.int32)] * 4,
        [pltpu.VMEM((CH,), jnp.int32)] * 4,
        pltpu.VMEM_SHARED((N, D), jnp.float32),
        [pltpu.SemaphoreType.DMA] * 2,
        [pltpu.SemaphoreType.DMA] * 4,
        [pltpu.SemaphoreType.DMA] * 2,
    ],
    compiler_params=pltpu.CompilerParams(needs_layout_passes=False),
)
def _agg_sc(xm_hbm, t_hbm, row_hbm, col_hbm, pm_hbm,
            G, T_, IR, IC, sm, SI, SX, SS):
    cid = lax.axis_index("c")
    sid = lax.axis_index("s")
    wid = cid * NS + sid
    base0 = wid * EPW

    def fill_zb(i, c2):
        for j in range(D // 16):
            G[0][i, pl.ds(16 * j, 16)] = jnp.zeros((16,), jnp.float32)
        return c2

    lax.fori_loop(0, CH, fill_zb, 0)
    for r in range(CPR // CH):
        pltpu.sync_copy(G[0], sm.at[pl.ds(sid * CPR + r * CH, CH)])
    pltpu.sync_copy(G[0].at[pl.ds(0, CPR - (CPR // CH) * CH)],
                    sm.at[pl.ds(sid * CPR + (CPR // CH) * CH,
                                CPR - (CPR // CH) * CH)])

    @pl.when(sid == NS - 1)
    def _():
        pltpu.sync_copy(G[0].at[pl.ds(0, N - NS * CPR)],
                        sm.at[pl.ds(NS * CPR, N - NS * CPR)])

    plsc.subcore_barrier()

    def idx_descs(k, slot):
        base = base0 + k * CH
        return (
            pltpu.make_async_copy(row_hbm.at[pl.ds(base, CH)], IR[slot],
                                  SX[slot]),
            pltpu.make_async_copy(col_hbm.at[pl.ds(base, CH)], IC[slot],
                                  SX[slot]),
        )

    def gt_descs(k, s, slot):
        base = base0 + k * CH
        return (
            pltpu.make_async_copy(xm_hbm.at[IR[slot]], G[s], SI[s]),
            pltpu.make_async_copy(t_hbm.at[pl.ds(base, CH)], T_[s], SI[s]),
        )

    def scat_desc(s, slot):
        return pltpu.make_async_copy(G[s], sm.at[IC[slot]], SS[s])

    def step(k, s, slot):
        for dsc in gt_descs(k, s, slot):
            dsc.wait()

        def body(i, c2):
            for j in range(D // 16):
                sl = pl.ds(16 * j, 16)
                G[s][i, sl] = jnp.maximum(G[s][i, sl] + T_[s][i, sl], 0.0)
            return c2

        lax.fori_loop(0, CH, body, 0)
        pltpu.async_copy(G[s], sm.at[IC[slot]], SS[s], add=True)

    for dsc in idx_descs(0, 0):
        dsc.start()
    for dsc in idx_descs(1, 1):
        dsc.start()
    for dsc in idx_descs(0, 0):
        dsc.wait()
    for dsc in gt_descs(0, 0, 0):
        dsc.start()

    @pl.loop(0, NIT - 1, step=4)
    def _(kk):
        for b2 in range(4):
            k = kk + b2
            s = b2 % 2
            slot = b2
            nslot = (b2 + 1) % 4
            pslot = (b2 + 2) % 4

            @pl.when(k >= 1)
            def _():
                scat_desc(1 - s, (b2 + 3) % 4).wait()

            @pl.when(k <= NIT - 3)
            def _():
                for dsc in idx_descs(k + 2, pslot):
                    dsc.start()

            for dsc in idx_descs(k + 1, nslot):
                dsc.wait()
            for dsc in gt_descs(k + 1, 1 - s, nslot):
                dsc.start()

            step(k, s, slot)

    scat_desc(1, 3).wait()
    step(NIT - 1, 0, 0)
    scat_desc(0, 0).wait()

    plsc.subcore_barrier()

    pltpu.sync_copy(sm.at[pl.ds(sid * CPR, CPR)],
                    pm_hbm.at[cid, pl.ds(sid * CPR, CPR)])

    @pl.when(sid == NS - 1)
    def _():
        pltpu.sync_copy(sm.at[pl.ds(NS * CPR, N - NS * CPR)],
                        pm_hbm.at[cid, pl.ds(NS * CPR, N - NS * CPR)])


def kernel(x, edge_index, edge_attr, u, W_edge, b_edge, W_node1, b_node1,
           W_node2, b_node2, W_glob, b_glob):
    f32 = jnp.float32

    W_esrc = W_edge[:D]
    W_edst = W_edge[D:2 * D]
    W_ee = W_edge[2 * D:2 * D + DE]
    W_eu = W_edge[2 * D + DE:]
    W1x = W_node1[:D]
    W1e = W_node1[D:]
    W2x = W_node2[:D]
    W2a = W_node2[D:2 * D]
    W2u = W_node2[2 * D:]
    Wg_u = W_glob[:DU]
    Wg_x = W_glob[DU:]
    be2 = b_edge.reshape(1, DE)
    b12 = b_node1.reshape(1, D)
    b22 = b_node2.reshape(1, D)
    bg2 = b_glob.reshape(1, DU)

    NB = 5
    NBR = N // NB
    EB = 80
    EBR = E // EB

    Wsd = jnp.concatenate(
        [W_esrc, W_edst, jnp.zeros((D, D - 2 * DE), f32)], axis=1)
    xsd, xm = pl.pallas_call(
        _proj_body,
        grid=(NB,),
        in_specs=[
            pl.BlockSpec((NBR, D), lambda i: (i, 0)),
            pl.BlockSpec((D, D), lambda i: (0, 0)),
            pl.BlockSpec((D, D), lambda i: (0, 0)),
        ],
        out_specs=[
            pl.BlockSpec((NBR, D), lambda i: (i, 0)),
            pl.BlockSpec((NBR, D), lambda i: (i, 0)),
        ],
        out_shape=[
            jax.ShapeDtypeStruct((N, D), f32),
            jax.ShapeDtypeStruct((N, D), f32),
        ],
    )(x, Wsd, W1x)

    c = pl.pallas_call(
        _edgeconst_body,
        grid=(EB,),
        in_specs=[
            pl.BlockSpec((EBR, DE), lambda i: (i, 0)),
            pl.BlockSpec((DE, DE), lambda i: (0, 0)),
            pl.BlockSpec((1, DU), lambda i: (0, 0)),
            pl.BlockSpec((DU, DE), lambda i: (0, 0)),
            pl.BlockSpec((1, DE), lambda i: (0, 0)),
        ],
        out_specs=pl.BlockSpec((EBR, DE), lambda i: (i, 0)),
        out_shape=jax.ShapeDtypeStruct((E, DE), f32),
    )(edge_attr, W_ee, u, W_eu, be2)

    row = edge_index[0]
    col = edge_index[1]

    enew, pc = _edge_sc(xsd, c, row, col)

    t = pl.pallas_call(
        _tmat_body,
        grid=(EB,),
        in_specs=[
            pl.BlockSpec((EBR, DE), lambda i: (i, 0)),
            pl.BlockSpec((DE, D), lambda i: (0, 0)),
            pl.BlockSpec((1, D), lambda i: (0, 0)),
        ],
        out_specs=pl.BlockSpec((EBR, D), lambda i: (i, 0)),
        out_shape=jax.ShapeDtypeStruct((E, D), f32),
    )(enew, W1e, b12)

    pm = _agg_sc(xm, t, row, col)

    pct = pc.reshape(NW, N).T
    new_x, new_u = pl.pallas_call(
        _node_body,
        grid=(NB,),
        in_specs=[
            pl.BlockSpec((NBR, D), lambda i: (i, 0)),
            pl.BlockSpec((NBR, D), lambda i: (i, 0)),
            pl.BlockSpec((NBR, D), lambda i: (i, 0)),
            pl.BlockSpec((NBR, NW), lambda i: (i, 0)),
            pl.BlockSpec((1, DU), lambda i: (0, 0)),
            pl.BlockSpec((D, D), lambda i: (0, 0)),
            pl.BlockSpec((D, D), lambda i: (0, 0)),
            pl.BlockSpec((DU, D), lambda i: (0, 0)),
            pl.BlockSpec((1, D), lambda i: (0, 0)),
            pl.BlockSpec((DU, DU), lambda i: (0, 0)),
            pl.BlockSpec((D, DU), lambda i: (0, 0)),
            pl.BlockSpec((1, DU), lambda i: (0, 0)),
        ],
        out_specs=[
            pl.BlockSpec((NBR, D), lambda i: (i, 0)),
            pl.BlockSpec((1, DU), lambda i: (0, 0)),
        ],
        out_shape=[
            jax.ShapeDtypeStruct((N, D), f32),
            jax.ShapeDtypeStruct((1, DU), f32),
        ],
        scratch_shapes=[pltpu.VMEM((1, D), f32)],
    )(x, pm[0], pm[1], pct, u, W2x, W2a, W2u, b22, Wg_u, Wg_x, bg2)

    return (new_x, edge_index, enew, new_u)


# speedup vs baseline: 1.1087x; 1.0015x over previous
import functools

import jax
import jax.numpy as jnp
from jax import lax
from jax.experimental import pallas as pl
from jax.experimental.pallas import tpu as pltpu
from jax.experimental.pallas import tpu_sc as plsc

N = 10000
E = 320000
D = 128
DE = 16
DU = 32

NC = 2
NS = 16
NW = NC * NS
EPW = E // NW
CH = 80
NIT = EPW // CH
CPR = 624


def _proj_body(x_ref, wsd_ref, wm_ref, xsd_ref, xm_ref):
    xb = x_ref[...]
    xsd_ref[...] = jnp.dot(xb, wsd_ref[...], preferred_element_type=jnp.float32)
    xm_ref[...] = jnp.dot(xb, wm_ref[...], preferred_element_type=jnp.float32)


def _edgeconst_body(ea_ref, wee_ref, u_ref, weu_ref, be_ref, c_ref):
    cu = jnp.dot(u_ref[...], weu_ref[...], preferred_element_type=jnp.float32)
    c_ref[...] = (jnp.dot(ea_ref[...], wee_ref[...],
                          preferred_element_type=jnp.float32) + cu + be_ref[...])


def _tmat_body(e_ref, w_ref, b_ref, t_ref):
    t_ref[...] = jnp.dot(e_ref[...], w_ref[...],
                         preferred_element_type=jnp.float32) + b_ref[...]


def _node_body(x_ref, p0m_ref, p1m_ref, pc_ref, u_ref,
               w2x_ref, w2a_ref, w2u_ref, b2_ref,
               wgu_ref, wgx_ref, bg_ref, nx_ref, nu_ref, s_ref):
    i = pl.program_id(0)
    cnt = jnp.maximum(jnp.sum(pc_ref[...], axis=1, keepdims=True), 1.0)
    agg = (p0m_ref[...] + p1m_ref[...]) / cnt
    nx = jnp.dot(x_ref[...], w2x_ref[...], preferred_element_type=jnp.float32)
    nx = nx + jnp.dot(agg, w2a_ref[...], preferred_element_type=jnp.float32)
    nx = nx + jnp.dot(u_ref[...], w2u_ref[...], preferred_element_type=jnp.float32)
    nx = jnp.maximum(nx + b2_ref[...], 0.0)
    nx_ref[...] = nx

    @pl.when(i == 0)
    def _():
        s_ref[...] = jnp.zeros_like(s_ref)

    s_ref[...] += jnp.sum(nx, axis=0, keepdims=True)

    @pl.when(i == pl.num_programs(0) - 1)
    def _():
        m = s_ref[...] * (1.0 / N)
        g = (jnp.dot(u_ref[...], wgu_ref[...],
                     preferred_element_type=jnp.float32)
             + jnp.dot(m, wgx_ref[...], preferred_element_type=jnp.float32)
             + bg_ref[...])
        nu_ref[...] = jnp.maximum(g, 0.0)


_MESH = plsc.VectorSubcoreMesh(core_axis_name="c", subcore_axis_name="s",
                               num_cores=NC, num_subcores=NS)


@functools.partial(
    pl.kernel,
    out_type=(jax.ShapeDtypeStruct((E, DE), jnp.float32),
              jax.ShapeDtypeStruct((NW * N,), jnp.float32)),
    mesh=_MESH,
    scratch_types=[
        pltpu.VMEM((EPW,), jnp.int32),
        pltpu.VMEM((EPW,), jnp.int32),
        [pltpu.VMEM((CH, D), jnp.float32)] * 2,
        [pltpu.VMEM((CH, D), jnp.float32)] * 2,
        [pltpu.VMEM((CH, DE), jnp.float32)] * 2,
        [pltpu.VMEM((CH, DE), jnp.float32)] * 2,
        pltpu.VMEM((N,), jnp.float32),
        [pltpu.SemaphoreType.DMA] * 2,
        [pltpu.SemaphoreType.DMA] * 2,
    ],
    compiler_params=pltpu.CompilerParams(needs_layout_passes=False),
)
def _edge_sc(xsd_hbm, c_hbm, row_hbm, col_hbm, enew_hbm, pc_hbm,
             idx_ra, idx_ca, A, B, C_, O, cnt_v, SI, SO):
    wid = lax.axis_index("c") * NS + lax.axis_index("s")
    base0 = wid * EPW

    def fill_zc(i, c2):
        cnt_v[pl.ds(16 * i, 16)] = jnp.zeros((16,), jnp.float32)
        return c2

    lax.fori_loop(0, N // 16, fill_zc, 0)

    pltpu.sync_copy(row_hbm.at[pl.ds(base0, EPW)], idx_ra)
    pltpu.sync_copy(col_hbm.at[pl.ds(base0, EPW)], idx_ca)

    def in_descs(k, s):
        off = k * CH
        return (
            pltpu.make_async_copy(xsd_hbm.at[idx_ra.at[pl.ds(off, CH)]],
                                  A[s], SI[s]),
            pltpu.make_async_copy(xsd_hbm.at[idx_ca.at[pl.ds(off, CH)]],
                                  B[s], SI[s]),
            pltpu.make_async_copy(c_hbm.at[pl.ds(base0 + off, CH)],
                                  C_[s], SI[s]),
        )

    def out_desc(k, s):
        return pltpu.make_async_copy(
            O[s], enew_hbm.at[pl.ds(base0 + k * CH, CH)], SO[s])

    def process(k, s):
        for dsc in in_descs(k, s):
            dsc.wait()

        def body(i, c2):
            O[s][i] = jnp.maximum(
                A[s][i, pl.ds(0, 16)] + B[s][i, pl.ds(16, 16)] + C_[s][i], 0.0)
            return c2

        lax.fori_loop(0, CH, body, 0)
        out_desc(k, s).start()

    for dsc in in_descs(0, 0):
        dsc.start()

    @pl.loop(0, NIT - 1, step=2)
    def _(kk):
        for b2 in range(2):
            k = kk + b2
            s = b2
            for dsc in in_descs(k + 1, 1 - s):
                dsc.start()

            @pl.when(k >= 2)
            def _():
                out_desc(k, s).wait()

            process(k, s)

    out_desc(NIT - 1, 0).wait()
    process(NIT - 1, 0)
    out_desc(NIT - 2, 1).wait()
    out_desc(NIT - 1, 0).wait()

    lanes = lax.iota(jnp.int32, 16)
    one16 = jnp.ones((16,), jnp.float32)

    def count(q, c2):
        idx16 = idx_ca[pl.ds(16 * q, 16)]
        for j in range(16):
            plsc.addupdate_scatter(cnt_v, [idx16], one16, mask=lanes == j)
        return c2

    lax.fori_loop(0, EPW // 16, count, 0)
    pltpu.sync_copy(cnt_v, pc_hbm.at[pl.ds(wid * N, N)])


@functools.partial(
    pl.kernel,
    out_type=jax.ShapeDtypeStruct((NC, N, D), jnp.float32),
    mesh=_MESH,
    scratch_types=[
        [pltpu.VMEM((CH, D), jnp.float32)] * 2,
        [pltpu.VMEM((CH, D), jnp.float32)] * 2,
        [pltpu.VMEM((CH,), jnp.int32)] * 4,
        [pltpu.VMEM((CH,), jnp.int32)] * 4,
        pltpu.VMEM_SHARED((N, D), jnp.float32),
        [pltpu.SemaphoreType.DMA] * 2,
        [pltpu.SemaphoreType.DMA] * 4,
        [pltpu.SemaphoreType.DMA] * 2,
    ],
    compiler_params=pltpu.CompilerParams(needs_layout_passes=False),
)
def _agg_sc(xm_hbm, t_hbm, row_hbm, col_hbm, pm_hbm,
            G, T_, IR, IC, sm, SI, SX, SS):
    cid = lax.axis_index("c")
    sid = lax.axis_index("s")
    wid = cid * NS + sid
    base0 = wid * EPW

    def fill_zb(i, c2):
        for j in range(D // 16):
            G[0][i, pl.ds(16 * j, 16)] = jnp.zeros((16,), jnp.float32)
        return c2

    lax.fori_loop(0, CH, fill_zb, 0)
    for r in range(CPR // CH):
        pltpu.sync_copy(G[0], sm.at[pl.ds(sid * CPR + r * CH, CH)])
    pltpu.sync_copy(G[0].at[pl.ds(0, CPR - (CPR // CH) * CH)],
                    sm.at[pl.ds(sid * CPR + (CPR // CH) * CH,
                                CPR - (CPR // CH) * CH)])

    @pl.when(sid == NS - 1)
    def _():
        pltpu.sync_copy(G[0].at[pl.ds(0, N - NS * CPR)],
                        sm.at[pl.ds(NS * CPR, N - NS * CPR)])

    plsc.subcore_barrier()

    def idx_descs(k, slot):
        base = base0 + k * CH
        return (
            pltpu.make_async_copy(row_hbm.at[pl.ds(base, CH)], IR[slot],
                                  SX[slot]),
            pltpu.make_async_copy(col_hbm.at[pl.ds(base, CH)], IC[slot],
                                  SX[slot]),
        )

    def gt_descs(k, s, slot):
        base = base0 + k * CH
        return (
            pltpu.make_async_copy(xm_hbm.at[IR[slot]], G[s], SI[s]),
            pltpu.make_async_copy(t_hbm.at[pl.ds(base, CH)], T_[s], SI[s]),
        )

    def scat_desc(s, slot):
        return pltpu.make_async_copy(G[s], sm.at[IC[slot]], SS[s])

    def step(k, s, slot):
        for dsc in gt_descs(k, s, slot):
            dsc.wait()

        def body(i, c2):
            for j in range(D // 16):
                sl = pl.ds(16 * j, 16)
                G[s][i, sl] = jnp.maximum(G[s][i, sl] + T_[s][i, sl], 0.0)
            return c2

        lax.fori_loop(0, CH, body, 0)
        pltpu.async_copy(G[s], sm.at[IC[slot]], SS[s], add=True)

    for dsc in idx_descs(0, 0):
        dsc.start()
    for dsc in idx_descs(1, 1):
        dsc.start()
    for dsc in idx_descs(0, 0):
        dsc.wait()
    for dsc in gt_descs(0, 0, 0):
        dsc.start()

    @pl.loop(0, NIT - 1, step=4)
    def _(kk):
        for b2 in range(4):
            k = kk + b2
            s = b2 % 2
            slot = b2
            nslot = (b2 + 1) % 4
            pslot = (b2 + 2) % 4

            @pl.when(k >= 1)
            def _():
                scat_desc(1 - s, (b2 + 3) % 4).wait()

            @pl.when(k <= NIT - 3)
            def _():
                for dsc in idx_descs(k + 2, pslot):
                    dsc.start()

            for dsc in idx_descs(k + 1, nslot):
                dsc.wait()
            for dsc in gt_descs(k + 1, 1 - s, nslot):
                dsc.start()

            step(k, s, slot)

    scat_desc(1, 3).wait()
    step(NIT - 1, 0, 0)
    scat_desc(0, 0).wait()

    plsc.subcore_barrier()

    pltpu.sync_copy(sm.at[pl.ds(sid * CPR, CPR)],
                    pm_hbm.at[cid, pl.ds(sid * CPR, CPR)])

    @pl.when(sid == NS - 1)
    def _():
        pltpu.sync_copy(sm.at[pl.ds(NS * CPR, N - NS * CPR)],
                        pm_hbm.at[cid, pl.ds(NS * CPR, N - NS * CPR)])


def kernel(x, edge_index, edge_attr, u, W_edge, b_edge, W_node1, b_node1,
           W_node2, b_node2, W_glob, b_glob):
    f32 = jnp.float32

    W_esrc = W_edge[:D]
    W_edst = W_edge[D:2 * D]
    W_ee = W_edge[2 * D:2 * D + DE]
    W_eu = W_edge[2 * D + DE:]
    W1x = W_node1[:D]
    W1e = W_node1[D:]
    W2x = W_node2[:D]
    W2a = W_node2[D:2 * D]
    W2u = W_node2[2 * D:]
    Wg_u = W_glob[:DU]
    Wg_x = W_glob[DU:]
    be2 = b_edge.reshape(1, DE)
    b12 = b_node1.reshape(1, D)
    b22 = b_node2.reshape(1, D)
    bg2 = b_glob.reshape(1, DU)

    NB = 5
    NBR = N // NB
    EB = 80
    EBR = E // EB

    Wsd = jnp.concatenate(
        [W_esrc, W_edst, jnp.zeros((D, D - 2 * DE), f32)], axis=1)
    xsd, xm = pl.pallas_call(
        _proj_body,
        grid=(NB,),
        in_specs=[
            pl.BlockSpec((NBR, D), lambda i: (i, 0)),
            pl.BlockSpec((D, D), lambda i: (0, 0)),
            pl.BlockSpec((D, D), lambda i: (0, 0)),
        ],
        out_specs=[
            pl.BlockSpec((NBR, D), lambda i: (i, 0)),
            pl.BlockSpec((NBR, D), lambda i: (i, 0)),
        ],
        out_shape=[
            jax.ShapeDtypeStruct((N, D), f32),
            jax.ShapeDtypeStruct((N, D), f32),
        ],
    )(x, Wsd, W1x)

    c = pl.pallas_call(
        _edgeconst_body,
        grid=(EB,),
        in_specs=[
            pl.BlockSpec((EBR, DE), lambda i: (i, 0)),
            pl.BlockSpec((DE, DE), lambda i: (0, 0)),
            pl.BlockSpec((1, DU), lambda i: (0, 0)),
            pl.BlockSpec((DU, DE), lambda i: (0, 0)),
            pl.BlockSpec((1, DE), lambda i: (0, 0)),
        ],
        out_specs=pl.BlockSpec((EBR, DE), lambda i: (i, 0)),
        out_shape=jax.ShapeDtypeStruct((E, DE), f32),
    )(edge_attr, W_ee, u, W_eu, be2)

    row = edge_index[0]
    col = edge_index[1]

    enew, pc = _edge_sc(xsd, c, row, col)

    t = pl.pallas_call(
        _tmat_body,
        grid=(EB,),
        in_specs=[
            pl.BlockSpec((EBR, DE), lambda i: (i, 0)),
            pl.BlockSpec((DE, D), lambda i: (0, 0)),
            pl.BlockSpec((1, D), lambda i: (0, 0)),
        ],
        out_specs=pl.BlockSpec((EBR, D), lambda i: (i, 0)),
        out_shape=jax.ShapeDtypeStruct((E, D), f32),
    )(enew, W1e, b12)

    pm = _agg_sc(xm, t, row, col)

    pct = pc.reshape(NW, N).T
    new_x, new_u = pl.pallas_call(
        _node_body,
        grid=(NB,),
        in_specs=[
            pl.BlockSpec((NBR, D), lambda i: (i, 0)),
            pl.BlockSpec((NBR, D), lambda i: (i, 0)),
            pl.BlockSpec((NBR, D), lambda i: (i, 0)),
            pl.BlockSpec((NBR, NW), lambda i: (i, 0)),
            pl.BlockSpec((1, DU), lambda i: (0, 0)),
            pl.BlockSpec((D, D), lambda i: (0, 0)),
            pl.BlockSpec((D, D), lambda i: (0, 0)),
            pl.BlockSpec((DU, D), lambda i: (0, 0)),
            pl.BlockSpec((1, D), lambda i: (0, 0)),
            pl.BlockSpec((DU, DU), lambda i: (0, 0)),
            pl.BlockSpec((D, DU), lambda i: (0, 0)),
            pl.BlockSpec((1, DU), lambda i: (0, 0)),
        ],
        out_specs=[
            pl.BlockSpec((NBR, D), lambda i: (i, 0)),
            pl.BlockSpec((1, DU), lambda i: (0, 0)),
        ],
        out_shape=[
            jax.ShapeDtypeStruct((N, D), f32),
            jax.ShapeDtypeStruct((1, DU), f32),
        ],
        scratch_shapes=[pltpu.VMEM((1, D), f32)],
    )(x, pm[0], pm[1], pct, u, W2x, W2a, W2u, b22, Wg_u, Wg_x, bg2)

    return (new_x, edge_index, enew, new_u)
